# Initial kernel scaffold; baseline (speedup 1.0000x reference)
import functools

import jax
import jax.numpy as jnp
from jax import lax
from jax.experimental import pallas as pl
from jax.experimental.pallas import tpu as pltpu
from jax.experimental.pallas import tpu_sc as plsc

f32 = jnp.float32

_N = 10000
_E = 320000
_D = 128

_NC = 2
_NS = 16
_NW = _NC * _NS
_EPW = _E // _NW
_CHUNK = 125
_NCHUNK = _EPW // _CHUNK
_RPS = _N // _NS
_RCH = 125
_NRC = _RPS // _RCH

_mesh = plsc.VectorSubcoreMesh(core_axis_name="c", subcore_axis_name="s")


@functools.partial(
    pl.kernel,
    out_type=jax.ShapeDtypeStruct((_NC, _N, 16), f32),
    mesh=_mesh,
    scratch_types=[
        pltpu.VMEM((_NCHUNK, _CHUNK), jnp.int32),
        pltpu.VMEM((_CHUNK, 16), f32),
        pltpu.VMEM((_RPS, 16), f32),
        pltpu.VMEM_SHARED((_N, 16), f32),
    ],
)
def _deg_kernel(dst_hbm, out_hbm, didx, ones, zbuf, acc):
    cid = lax.axis_index("c")
    sid = lax.axis_index("s")
    w = cid * _NS + sid

    @pl.loop(0, _CHUNK)
    def _(i):
        ones[i, :] = jnp.ones((16,), f32)

    @pl.loop(0, _RPS)
    def _(i):
        zbuf[i, :] = jnp.zeros((16,), f32)

    pltpu.sync_copy(zbuf, acc.at[pl.ds(sid * _RPS, _RPS)])
    plsc.subcore_barrier()

    pltpu.sync_copy(dst_hbm.at[w], didx)

    @pl.loop(0, _NCHUNK)
    def _(j):
        pltpu.sync_copy(ones, acc.at[didx.at[j]], add=True)

    plsc.subcore_barrier()
    pltpu.sync_copy(
        acc.at[pl.ds(sid * _RPS, _RPS)],
        out_hbm.at[cid, pl.ds(sid * _RPS, _RPS)],
    )


@functools.partial(
    pl.kernel,
    out_type=jax.ShapeDtypeStruct((_NC, _N, _D), f32),
    mesh=_mesh,
    scratch_types=[
        pltpu.VMEM((_NCHUNK, _CHUNK), jnp.int32),
        pltpu.VMEM((_NCHUNK, _CHUNK), jnp.int32),
        pltpu.VMEM((_CHUNK, _D), f32),
        pltpu.VMEM_SHARED((_N, _D), f32),
    ],
)
def _edge_kernel(y_hbm, src_hbm, dst_hbm, out_hbm, sidx, didx, rows, acc):
    cid = lax.axis_index("c")
    sid = lax.axis_index("s")
    w = cid * _NS + sid

    @pl.when(cid == 0)
    def _():
        @pl.loop(0, _NRC)
        def _(k):
            base = sid * _RPS + k * _RCH
            pltpu.sync_copy(y_hbm.at[pl.ds(base, _RCH)], acc.at[pl.ds(base, _RCH)])

    @pl.when(cid != 0)
    def _():
        @pl.loop(0, _CHUNK)
        def _(i):
            @pl.loop(0, _D // 16)
            def _(c):
                rows[i, pl.ds(c * 16, 16)] = jnp.zeros((16,), f32)

        @pl.loop(0, _NRC)
        def _(k):
            base = sid * _RPS + k * _RCH
            pltpu.sync_copy(rows.at[pl.ds(0, _RCH)], acc.at[pl.ds(base, _RCH)])

    plsc.subcore_barrier()

    pltpu.sync_copy(src_hbm.at[w], sidx)
    pltpu.sync_copy(dst_hbm.at[w], didx)

    @pl.loop(0, _NCHUNK)
    def _(j):
        pltpu.sync_copy(y_hbm.at[sidx.at[j]], rows)
        pltpu.sync_copy(rows, acc.at[didx.at[j]], add=True)

    plsc.subcore_barrier()

    @pl.loop(0, _NRC)
    def _(k):
        base = sid * _RPS + k * _RCH
        pltpu.sync_copy(
            acc.at[pl.ds(base, _RCH)], out_hbm.at[cid, pl.ds(base, _RCH)]
        )


_BLK = 1000
_GRID = _N // _BLK


def _mm(x, W):
    def body(x_ref, w_ref, o_ref):
        o_ref[...] = jnp.dot(
            x_ref[...], w_ref[...],
            preferred_element_type=f32, precision=lax.Precision.HIGHEST,
        )

    return pl.pallas_call(
        body,
        grid=(_GRID,),
        in_specs=[
            pl.BlockSpec((_BLK, _D), lambda i: (i, 0)),
            pl.BlockSpec((_D, _D), lambda i: (0, 0)),
        ],
        out_specs=pl.BlockSpec((_BLK, _D), lambda i: (i, 0)),
        out_shape=jax.ShapeDtypeStruct((_N, _D), f32),
    )(x, W)


def _scale(xw, da, db):
    def body(xw_ref, da_ref, db_ref, o_ref):
        d = lax.rsqrt(da_ref[:, 0:1] + db_ref[:, 0:1] + 1.0)
        o_ref[...] = xw_ref[...] * d

    return pl.pallas_call(
        body,
        grid=(_GRID,),
        in_specs=[
            pl.BlockSpec((_BLK, _D), lambda i: (i, 0)),
            pl.BlockSpec((_BLK, 16), lambda i: (i, 0)),
            pl.BlockSpec((_BLK, 16), lambda i: (i, 0)),
        ],
        out_specs=pl.BlockSpec((_BLK, _D), lambda i: (i, 0)),
        out_shape=jax.ShapeDtypeStruct((_N, _D), f32),
    )(xw, da, db)


def _mid(sa, sb, da, db, b1, W2):
    def body(sa_ref, sb_ref, da_ref, db_ref, b_ref, w_ref, o_ref):
        d = lax.rsqrt(da_ref[:, 0:1] + db_ref[:, 0:1] + 1.0)
        h = jnp.maximum(d * (sa_ref[...] + sb_ref[...]) + b_ref[...], 0.0)
        o_ref[...] = (
            jnp.dot(h, w_ref[...],
                    preferred_element_type=f32,
                    precision=lax.Precision.HIGHEST)
            * d
        )

    return pl.pallas_call(
        body,
        grid=(_GRID,),
        in_specs=[
            pl.BlockSpec((_BLK, _D), lambda i: (i, 0)),
            pl.BlockSpec((_BLK, _D), lambda i: (i, 0)),
            pl.BlockSpec((_BLK, 16), lambda i: (i, 0)),
            pl.BlockSpec((_BLK, 16), lambda i: (i, 0)),
            pl.BlockSpec((1, _D), lambda i: (0, 0)),
            pl.BlockSpec((_D, _D), lambda i: (0, 0)),
        ],
        out_specs=pl.BlockSpec((_BLK, _D), lambda i: (i, 0)),
        out_shape=jax.ShapeDtypeStruct((_N, _D), f32),
    )(sa, sb, da, db, b1, W2)


def _final(sa, sb, da, db, b2):
    def body(sa_ref, sb_ref, da_ref, db_ref, b_ref, o_ref):
        d = lax.rsqrt(da_ref[:, 0:1] + db_ref[:, 0:1] + 1.0)
        o_ref[...] = d * (sa_ref[...] + sb_ref[...]) + b_ref[...]

    return pl.pallas_call(
        body,
        grid=(_GRID,),
        in_specs=[
            pl.BlockSpec((_BLK, _D), lambda i: (i, 0)),
            pl.BlockSpec((_BLK, _D), lambda i: (i, 0)),
            pl.BlockSpec((_BLK, 16), lambda i: (i, 0)),
            pl.BlockSpec((_BLK, 16), lambda i: (i, 0)),
            pl.BlockSpec((1, _D), lambda i: (0, 0)),
        ],
        out_specs=pl.BlockSpec((_BLK, _D), lambda i: (i, 0)),
        out_shape=jax.ShapeDtypeStruct((_N, _D), f32),
    )(sa, sb, da, db, b2)


def kernel(x, edge_index, W1, b1, W2, b2):
    src3 = edge_index[0].reshape(_NW, _NCHUNK, _CHUNK)
    dst3 = edge_index[1].reshape(_NW, _NCHUNK, _CHUNK)

    degp = _deg_kernel(dst3)
    da, db = degp[0], degp[1]

    xw1 = _mm(x, W1)
    y1 = _scale(xw1, da, db)
    s1 = _edge_kernel(y1, src3, dst3)
    y2 = _mid(s1[0], s1[1], da, db, b1.reshape(1, _D), W2)
    s2 = _edge_kernel(y2, src3, dst3)
    return _final(s2[0], s2[1], da, db, b2.reshape(1, _D))


# speedup vs baseline: 19.6438x; 19.6438x over previous
import functools

import jax
import jax.numpy as jnp
from jax import lax
from jax.experimental import pallas as pl
from jax.experimental.pallas import tpu as pltpu
from jax.experimental.pallas import tpu_sc as plsc

f32 = jnp.float32

_N = 10000
_E = 320000
_D = 128

_NC = 2
_NS = 16
_NW = _NC * _NS
_EPW = _E // _NW
_CHUNK = 125
_NCHUNK = _EPW // _CHUNK
_RPS = 640
_RLAST = _N - 15 * _RPS
_ZCH = 80

_mesh = plsc.VectorSubcoreMesh(core_axis_name="c", subcore_axis_name="s")


@functools.partial(
    pl.kernel,
    out_type=jax.ShapeDtypeStruct((_NC, _N, _D), f32),
    mesh=_mesh,
    scratch_types=[
        pltpu.VMEM((_NCHUNK, _CHUNK), jnp.int32),
        pltpu.VMEM((_CHUNK, _D), f32),
        pltpu.VMEM((_ZCH, _D), f32),
        pltpu.VMEM_SHARED((_N, _D), f32),
    ],
)
def _deg_kernel(dst_hbm, out_hbm, didx, ones, zbuf, acc):
    cid = lax.axis_index("c")
    sid = lax.axis_index("s")
    w = cid * _NS + sid
    base = sid * _RPS

    @pl.loop(0, _CHUNK)
    def _(i):
        @pl.loop(0, _D // 16)
        def _(c):
            ones[i, pl.ds(c * 16, 16)] = jnp.ones((16,), f32)

    @pl.loop(0, _ZCH)
    def _(i):
        @pl.loop(0, _D // 16)
        def _(c):
            zbuf[i, pl.ds(c * 16, 16)] = jnp.zeros((16,), f32)

    @pl.when(sid < _NS - 1)
    def _():
        @pl.loop(0, _RPS // _ZCH)
        def _(k):
            pltpu.sync_copy(zbuf, acc.at[pl.ds(base + k * _ZCH, _ZCH)])

    @pl.when(sid == _NS - 1)
    def _():
        @pl.loop(0, _RLAST // _ZCH)
        def _(k):
            pltpu.sync_copy(zbuf, acc.at[pl.ds(base + k * _ZCH, _ZCH)])

    plsc.subcore_barrier()

    pltpu.sync_copy(dst_hbm.at[w], didx)

    @pl.loop(0, _NCHUNK)
    def _(j):
        pltpu.sync_copy(ones, acc.at[didx.at[j]], add=True)

    plsc.subcore_barrier()

    @pl.when(sid < _NS - 1)
    def _():
        pltpu.sync_copy(
            acc.at[pl.ds(base, _RPS)], out_hbm.at[cid, pl.ds(base, _RPS)]
        )

    @pl.when(sid == _NS - 1)
    def _():
        pltpu.sync_copy(
            acc.at[pl.ds(base, _RLAST)], out_hbm.at[cid, pl.ds(base, _RLAST)]
        )


@functools.partial(
    pl.kernel,
    out_type=jax.ShapeDtypeStruct((_NC, _N, _D), f32),
    mesh=_mesh,
    scratch_types=[
        pltpu.VMEM((_NCHUNK, _CHUNK), jnp.int32),
        pltpu.VMEM((_NCHUNK, _CHUNK), jnp.int32),
        pltpu.VMEM((_CHUNK, _D), f32),
        pltpu.VMEM_SHARED((_N, _D), f32),
    ],
)
def _edge_kernel(y_hbm, src_hbm, dst_hbm, out_hbm, sidx, didx, rows, acc):
    cid = lax.axis_index("c")
    sid = lax.axis_index("s")
    w = cid * _NS + sid
    base = sid * _RPS

    @pl.loop(0, _ZCH)
    def _(i):
        @pl.loop(0, _D // 16)
        def _(c):
            rows[i, pl.ds(c * 16, 16)] = jnp.zeros((16,), f32)

    zsrc = rows.at[pl.ds(0, _ZCH)]

    @pl.when(sid < _NS - 1)
    def _():
        @pl.loop(0, _RPS // _ZCH)
        def _(k):
            pltpu.sync_copy(zsrc, acc.at[pl.ds(base + k * _ZCH, _ZCH)])

    @pl.when(sid == _NS - 1)
    def _():
        @pl.loop(0, _RLAST // _ZCH)
        def _(k):
            pltpu.sync_copy(zsrc, acc.at[pl.ds(base + k * _ZCH, _ZCH)])

    plsc.subcore_barrier()

    pltpu.sync_copy(src_hbm.at[w], sidx)
    pltpu.sync_copy(dst_hbm.at[w], didx)

    @pl.loop(0, _NCHUNK)
    def _(j):
        pltpu.sync_copy(y_hbm.at[sidx.at[j]], rows)
        pltpu.sync_copy(rows, acc.at[didx.at[j]], add=True)

    plsc.subcore_barrier()

    @pl.when(sid < _NS - 1)
    def _():
        pltpu.sync_copy(
            acc.at[pl.ds(base, _RPS)], out_hbm.at[cid, pl.ds(base, _RPS)]
        )

    @pl.when(sid == _NS - 1)
    def _():
        pltpu.sync_copy(
            acc.at[pl.ds(base, _RLAST)], out_hbm.at[cid, pl.ds(base, _RLAST)]
        )


_BLK = 1000
_GRID = _N // _BLK


def _mm(x, W):
    def body(x_ref, w_ref, o_ref):
        o_ref[...] = jnp.dot(
            x_ref[...], w_ref[...],
            preferred_element_type=f32, precision=lax.Precision.HIGHEST,
        )

    return pl.pallas_call(
        body,
        grid=(_GRID,),
        in_specs=[
            pl.BlockSpec((_BLK, _D), lambda i: (i, 0)),
            pl.BlockSpec((_D, _D), lambda i: (0, 0)),
        ],
        out_specs=pl.BlockSpec((_BLK, _D), lambda i: (i, 0)),
        out_shape=jax.ShapeDtypeStruct((_N, _D), f32),
    )(x, W)


def _scale(xw, da, db):
    def body(xw_ref, da_ref, db_ref, o_ref):
        d = lax.rsqrt(da_ref[:, 0:1] + db_ref[:, 0:1] + 1.0)
        o_ref[...] = xw_ref[...] * d

    return pl.pallas_call(
        body,
        grid=(_GRID,),
        in_specs=[
            pl.BlockSpec((_BLK, _D), lambda i: (i, 0)),
            pl.BlockSpec((_BLK, _D), lambda i: (i, 0)),
            pl.BlockSpec((_BLK, _D), lambda i: (i, 0)),
        ],
        out_specs=pl.BlockSpec((_BLK, _D), lambda i: (i, 0)),
        out_shape=jax.ShapeDtypeStruct((_N, _D), f32),
    )(xw, da, db)


def _mid(sa, sb, y, da, db, b1, W2):
    def body(sa_ref, sb_ref, y_ref, da_ref, db_ref, b_ref, w_ref, o_ref):
        d = lax.rsqrt(da_ref[:, 0:1] + db_ref[:, 0:1] + 1.0)
        h = jnp.maximum(
            d * (sa_ref[...] + sb_ref[...] + y_ref[...]) + b_ref[...], 0.0
        )
        o_ref[...] = (
            jnp.dot(h, w_ref[...],
                    preferred_element_type=f32,
                    precision=lax.Precision.HIGHEST)
            * d
        )

    return pl.pallas_call(
        body,
        grid=(_GRID,),
        in_specs=[
            pl.BlockSpec((_BLK, _D), lambda i: (i, 0)),
            pl.BlockSpec((_BLK, _D), lambda i: (i, 0)),
            pl.BlockSpec((_BLK, _D), lambda i: (i, 0)),
            pl.BlockSpec((_BLK, _D), lambda i: (i, 0)),
            pl.BlockSpec((_BLK, _D), lambda i: (i, 0)),
            pl.BlockSpec((1, _D), lambda i: (0, 0)),
            pl.BlockSpec((_D, _D), lambda i: (0, 0)),
        ],
        out_specs=pl.BlockSpec((_BLK, _D), lambda i: (i, 0)),
        out_shape=jax.ShapeDtypeStruct((_N, _D), f32),
    )(sa, sb, y, da, db, b1, W2)


def _final(sa, sb, y, da, db, b2):
    def body(sa_ref, sb_ref, y_ref, da_ref, db_ref, b_ref, o_ref):
        d = lax.rsqrt(da_ref[:, 0:1] + db_ref[:, 0:1] + 1.0)
        o_ref[...] = d * (sa_ref[...] + sb_ref[...] + y_ref[...]) + b_ref[...]

    return pl.pallas_call(
        body,
        grid=(_GRID,),
        in_specs=[
            pl.BlockSpec((_BLK, _D), lambda i: (i, 0)),
            pl.BlockSpec((_BLK, _D), lambda i: (i, 0)),
            pl.BlockSpec((_BLK, _D), lambda i: (i, 0)),
            pl.BlockSpec((_BLK, _D), lambda i: (i, 0)),
            pl.BlockSpec((_BLK, _D), lambda i: (i, 0)),
            pl.BlockSpec((1, _D), lambda i: (0, 0)),
        ],
        out_specs=pl.BlockSpec((_BLK, _D), lambda i: (i, 0)),
        out_shape=jax.ShapeDtypeStruct((_N, _D), f32),
    )(sa, sb, y, da, db, b2)


def kernel(x, edge_index, W1, b1, W2, b2):
    src3 = edge_index[0].reshape(_NW, _NCHUNK, _CHUNK)
    dst3 = edge_index[1].reshape(_NW, _NCHUNK, _CHUNK)

    degp = _deg_kernel(dst3)
    da, db = degp[0], degp[1]

    xw1 = _mm(x, W1)
    y1 = _scale(xw1, da, db)
    s1 = _edge_kernel(y1, src3, dst3)
    y2 = _mid(s1[0], s1[1], y1, da, db, b1.reshape(1, _D), W2)
    s2 = _edge_kernel(y2, src3, dst3)
    return _final(s2[0], s2[1], y2, da, db, b2.reshape(1, _D))


# speedup vs baseline: 22.3438x; 1.1375x over previous
import functools

import jax
import jax.numpy as jnp
from jax import lax
from jax.experimental import pallas as pl
from jax.experimental.pallas import tpu as pltpu
from jax.experimental.pallas import tpu_sc as plsc

f32 = jnp.float32

_N = 10000
_E = 320000
_D = 128

_NC = 2
_NS = 16
_NW = _NC * _NS
_EPW = _E // _NW
_CHUNK = 125
_NCHUNK = _EPW // _CHUNK
_NHALF = 2
_HCH = _NCHUNK // _NHALF
_RPS = 640
_RLAST = _N - 15 * _RPS
_ZCH = 80

_mesh = plsc.VectorSubcoreMesh(core_axis_name="c", subcore_axis_name="s")


@functools.partial(
    pl.kernel,
    out_type=jax.ShapeDtypeStruct((_NC, _N, _D), f32),
    mesh=_mesh,
    scratch_types=[
        pltpu.VMEM((_NCHUNK, _CHUNK), jnp.int32),
        pltpu.VMEM((_CHUNK, _D), f32),
        pltpu.VMEM((_ZCH, _D), f32),
        pltpu.VMEM_SHARED((_N, _D), f32),
        pltpu.SemaphoreType.DMA,
    ],
)
def _deg_kernel(dst_hbm, out_hbm, didx, ones, zbuf, acc, ssem):
    cid = lax.axis_index("c")
    sid = lax.axis_index("s")
    w = cid * _NS + sid
    base = sid * _RPS

    @pl.loop(0, _CHUNK)
    def _(i):
        @pl.loop(0, _D // 16)
        def _(c):
            ones[i, pl.ds(c * 16, 16)] = jnp.ones((16,), f32)

    @pl.loop(0, _ZCH)
    def _(i):
        @pl.loop(0, _D // 16)
        def _(c):
            zbuf[i, pl.ds(c * 16, 16)] = jnp.zeros((16,), f32)

    @pl.when(sid < _NS - 1)
    def _():
        @pl.loop(0, _RPS // _ZCH)
        def _(k):
            pltpu.sync_copy(zbuf, acc.at[pl.ds(base + k * _ZCH, _ZCH)])

    @pl.when(sid == _NS - 1)
    def _():
        @pl.loop(0, _RLAST // _ZCH)
        def _(k):
            pltpu.sync_copy(zbuf, acc.at[pl.ds(base + k * _ZCH, _ZCH)])

    plsc.subcore_barrier()

    pltpu.sync_copy(dst_hbm.at[w], didx)

    @pl.loop(0, _NCHUNK)
    def _(j):
        pltpu.async_copy(ones, acc.at[didx.at[j]], ssem, add=True)

    @pl.loop(0, _NCHUNK)
    def _(j):
        pltpu.make_async_copy(ones, acc.at[didx.at[j]], ssem).wait()

    plsc.subcore_barrier()

    @pl.when(sid < _NS - 1)
    def _():
        pltpu.sync_copy(
            acc.at[pl.ds(base, _RPS)], out_hbm.at[cid, pl.ds(base, _RPS)]
        )

    @pl.when(sid == _NS - 1)
    def _():
        pltpu.sync_copy(
            acc.at[pl.ds(base, _RLAST)], out_hbm.at[cid, pl.ds(base, _RLAST)]
        )


@functools.partial(
    pl.kernel,
    out_type=jax.ShapeDtypeStruct((_NC, _N, _D), f32),
    mesh=_mesh,
    scratch_types=[
        pltpu.VMEM((_HCH, _CHUNK), jnp.int32),
        pltpu.VMEM((_HCH, _CHUNK), jnp.int32),
        pltpu.VMEM((_CHUNK, _D), f32),
        pltpu.VMEM((_CHUNK, _D), f32),
        pltpu.VMEM_SHARED((_N, _D), f32),
        pltpu.SemaphoreType.DMA,
        pltpu.SemaphoreType.DMA,
        pltpu.SemaphoreType.DMA,
        pltpu.SemaphoreType.DMA,
    ],
)
def _edge_kernel(y_hbm, src_hbm, dst_hbm, out_hbm, sidx, didx, rows, rows2,
                 acc, gsa, gsb, ssa, ssb):
    cid = lax.axis_index("c")
    sid = lax.axis_index("s")
    w = cid * _NS + sid
    base = sid * _RPS

    @pl.loop(0, _ZCH)
    def _(i):
        @pl.loop(0, _D // 16)
        def _(c):
            rows[i, pl.ds(c * 16, 16)] = jnp.zeros((16,), f32)

    zsrc = rows.at[pl.ds(0, _ZCH)]

    @pl.when(sid < _NS - 1)
    def _():
        @pl.loop(0, _RPS // _ZCH)
        def _(k):
            pltpu.sync_copy(zsrc, acc.at[pl.ds(base + k * _ZCH, _ZCH)])

    @pl.when(sid == _NS - 1)
    def _():
        @pl.loop(0, _RLAST // _ZCH)
        def _(k):
            pltpu.sync_copy(zsrc, acc.at[pl.ds(base + k * _ZCH, _ZCH)])

    plsc.subcore_barrier()

    for h in range(_NHALF):
        pltpu.sync_copy(src_hbm.at[w, pl.ds(h * _HCH, _HCH)], sidx)
        pltpu.sync_copy(dst_hbm.at[w, pl.ds(h * _HCH, _HCH)], didx)

        pltpu.async_copy(y_hbm.at[sidx.at[0]], rows, gsa)
        pltpu.async_copy(y_hbm.at[sidx.at[1]], rows2, gsb)

        @pl.loop(0, _HCH, step=2)
        def _(j):
            pltpu.make_async_copy(y_hbm.at[sidx.at[j]], rows, gsa).wait()
            pltpu.async_copy(rows, acc.at[didx.at[j]], ssa, add=True)
            pltpu.make_async_copy(y_hbm.at[sidx.at[j + 1]], rows2, gsb).wait()
            pltpu.async_copy(rows2, acc.at[didx.at[j + 1]], ssb, add=True)

            @pl.when(j + 2 < _HCH)
            def _():
                pltpu.make_async_copy(rows, acc.at[didx.at[j]], ssa).wait()
                pltpu.async_copy(y_hbm.at[sidx.at[j + 2]], rows, gsa)
                pltpu.make_async_copy(rows2, acc.at[didx.at[j + 1]], ssb).wait()
                pltpu.async_copy(y_hbm.at[sidx.at[j + 3]], rows2, gsb)

            @pl.when(j + 2 >= _HCH)
            def _():
                pltpu.make_async_copy(rows, acc.at[didx.at[j]], ssa).wait()
                pltpu.make_async_copy(rows2, acc.at[didx.at[j + 1]], ssb).wait()

    plsc.subcore_barrier()

    @pl.when(sid < _NS - 1)
    def _():
        pltpu.sync_copy(
            acc.at[pl.ds(base, _RPS)], out_hbm.at[cid, pl.ds(base, _RPS)]
        )

    @pl.when(sid == _NS - 1)
    def _():
        pltpu.sync_copy(
            acc.at[pl.ds(base, _RLAST)], out_hbm.at[cid, pl.ds(base, _RLAST)]
        )


_BLK = 1000
_GRID = _N // _BLK


def _mm(x, W):
    def body(x_ref, w_ref, o_ref):
        o_ref[...] = jnp.dot(
            x_ref[...], w_ref[...],
            preferred_element_type=f32, precision=lax.Precision.HIGHEST,
        )

    return pl.pallas_call(
        body,
        grid=(_GRID,),
        in_specs=[
            pl.BlockSpec((_BLK, _D), lambda i: (i, 0)),
            pl.BlockSpec((_D, _D), lambda i: (0, 0)),
        ],
        out_specs=pl.BlockSpec((_BLK, _D), lambda i: (i, 0)),
        out_shape=jax.ShapeDtypeStruct((_N, _D), f32),
    )(x, W)


def _scale(xw, da, db):
    def body(xw_ref, da_ref, db_ref, o_ref):
        d = lax.rsqrt(da_ref[:, 0:1] + db_ref[:, 0:1] + 1.0)
        o_ref[...] = xw_ref[...] * d

    return pl.pallas_call(
        body,
        grid=(_GRID,),
        in_specs=[
            pl.BlockSpec((_BLK, _D), lambda i: (i, 0)),
            pl.BlockSpec((_BLK, _D), lambda i: (i, 0)),
            pl.BlockSpec((_BLK, _D), lambda i: (i, 0)),
        ],
        out_specs=pl.BlockSpec((_BLK, _D), lambda i: (i, 0)),
        out_shape=jax.ShapeDtypeStruct((_N, _D), f32),
    )(xw, da, db)


def _mid(sa, sb, y, da, db, b1, W2):
    def body(sa_ref, sb_ref, y_ref, da_ref, db_ref, b_ref, w_ref, o_ref):
        d = lax.rsqrt(da_ref[:, 0:1] + db_ref[:, 0:1] + 1.0)
        h = jnp.maximum(
            d * (sa_ref[...] + sb_ref[...] + y_ref[...]) + b_ref[...], 0.0
        )
        o_ref[...] = (
            jnp.dot(h, w_ref[...],
                    preferred_element_type=f32,
                    precision=lax.Precision.HIGHEST)
            * d
        )

    return pl.pallas_call(
        body,
        grid=(_GRID,),
        in_specs=[
            pl.BlockSpec((_BLK, _D), lambda i: (i, 0)),
            pl.BlockSpec((_BLK, _D), lambda i: (i, 0)),
            pl.BlockSpec((_BLK, _D), lambda i: (i, 0)),
            pl.BlockSpec((_BLK, _D), lambda i: (i, 0)),
            pl.BlockSpec((_BLK, _D), lambda i: (i, 0)),
            pl.BlockSpec((1, _D), lambda i: (0, 0)),
            pl.BlockSpec((_D, _D), lambda i: (0, 0)),
        ],
        out_specs=pl.BlockSpec((_BLK, _D), lambda i: (i, 0)),
        out_shape=jax.ShapeDtypeStruct((_N, _D), f32),
    )(sa, sb, y, da, db, b1, W2)


def _final(sa, sb, y, da, db, b2):
    def body(sa_ref, sb_ref, y_ref, da_ref, db_ref, b_ref, o_ref):
        d = lax.rsqrt(da_ref[:, 0:1] + db_ref[:, 0:1] + 1.0)
        o_ref[...] = d * (sa_ref[...] + sb_ref[...] + y_ref[...]) + b_ref[...]

    return pl.pallas_call(
        body,
        grid=(_GRID,),
        in_specs=[
            pl.BlockSpec((_BLK, _D), lambda i: (i, 0)),
            pl.BlockSpec((_BLK, _D), lambda i: (i, 0)),
            pl.BlockSpec((_BLK, _D), lambda i: (i, 0)),
            pl.BlockSpec((_BLK, _D), lambda i: (i, 0)),
            pl.BlockSpec((_BLK, _D), lambda i: (i, 0)),
            pl.BlockSpec((1, _D), lambda i: (0, 0)),
        ],
        out_specs=pl.BlockSpec((_BLK, _D), lambda i: (i, 0)),
        out_shape=jax.ShapeDtypeStruct((_N, _D), f32),
    )(sa, sb, y, da, db, b2)


def kernel(x, edge_index, W1, b1, W2, b2):
    src3 = edge_index[0].reshape(_NW, _NCHUNK, _CHUNK)
    dst3 = edge_index[1].reshape(_NW, _NCHUNK, _CHUNK)

    degp = _deg_kernel(dst3)
    da, db = degp[0], degp[1]

    xw1 = _mm(x, W1)
    y1 = _scale(xw1, da, db)
    s1 = _edge_kernel(y1, src3, dst3)
    y2 = _mid(s1[0], s1[1], y1, da, db, b1.reshape(1, _D), W2)
    s2 = _edge_kernel(y2, src3, dst3)
    return _final(s2[0], s2[1], y2, da, db, b2.reshape(1, _D))


# speedup vs baseline: 24.1222x; 1.0796x over previous
import functools

import jax
import jax.numpy as jnp
from jax import lax
from jax.experimental import pallas as pl
from jax.experimental.pallas import tpu as pltpu
from jax.experimental.pallas import tpu_sc as plsc

f32 = jnp.float32

_N = 10000
_E = 320000
_D = 128

_NC = 2
_NS = 16
_NW = _NC * _NS
_EPW = _E // _NW
_CHUNK = 50
_NCHUNK = _EPW // _CHUNK
_NB = 4
_NSEC = 5
_SCH = _NCHUNK // _NSEC
_RPS = 640
_RLAST = _N - 15 * _RPS
_ZCH = 40

_mesh = plsc.VectorSubcoreMesh(core_axis_name="c", subcore_axis_name="s")


@functools.partial(
    pl.kernel,
    out_type=jax.ShapeDtypeStruct((_NC, _N, _D), f32),
    mesh=_mesh,
    scratch_types=[
        pltpu.VMEM((_NCHUNK, _CHUNK), jnp.int32),
        pltpu.VMEM((_CHUNK, _D), f32),
        pltpu.VMEM((_ZCH, _D), f32),
        pltpu.VMEM_SHARED((_N, _D), f32),
        pltpu.SemaphoreType.DMA,
    ],
)
def _deg_kernel(dst_hbm, out_hbm, didx, ones, zbuf, acc, ssem):
    cid = lax.axis_index("c")
    sid = lax.axis_index("s")
    w = cid * _NS + sid
    base = sid * _RPS

    @pl.loop(0, _CHUNK)
    def _(i):
        @pl.loop(0, _D // 16)
        def _(c):
            ones[i, pl.ds(c * 16, 16)] = jnp.ones((16,), f32)

    @pl.loop(0, _ZCH)
    def _(i):
        @pl.loop(0, _D // 16)
        def _(c):
            zbuf[i, pl.ds(c * 16, 16)] = jnp.zeros((16,), f32)

    @pl.when(sid < _NS - 1)
    def _():
        @pl.loop(0, _RPS // _ZCH)
        def _(k):
            pltpu.sync_copy(zbuf, acc.at[pl.ds(base + k * _ZCH, _ZCH)])

    @pl.when(sid == _NS - 1)
    def _():
        @pl.loop(0, _RLAST // _ZCH)
        def _(k):
            pltpu.sync_copy(zbuf, acc.at[pl.ds(base + k * _ZCH, _ZCH)])

    plsc.subcore_barrier()

    pltpu.sync_copy(dst_hbm.at[w], didx)

    @pl.loop(0, _NCHUNK)
    def _(j):
        pltpu.async_copy(ones, acc.at[didx.at[j]], ssem, add=True)

    @pl.loop(0, _NCHUNK)
    def _(j):
        pltpu.make_async_copy(ones, acc.at[didx.at[j]], ssem).wait()

    plsc.subcore_barrier()

    @pl.when(sid < _NS - 1)
    def _():
        pltpu.sync_copy(
            acc.at[pl.ds(base, _RPS)], out_hbm.at[cid, pl.ds(base, _RPS)]
        )

    @pl.when(sid == _NS - 1)
    def _():
        pltpu.sync_copy(
            acc.at[pl.ds(base, _RLAST)], out_hbm.at[cid, pl.ds(base, _RLAST)]
        )


@functools.partial(
    pl.kernel,
    out_type=jax.ShapeDtypeStruct((_NC, _N, _D), f32),
    mesh=_mesh,
    scratch_types=[
        pltpu.VMEM((_SCH, _CHUNK), jnp.int32),
        pltpu.VMEM((_SCH, _CHUNK), jnp.int32),
        pltpu.VMEM((_CHUNK, _D), f32),
        pltpu.VMEM((_CHUNK, _D), f32),
        pltpu.VMEM((_CHUNK, _D), f32),
        pltpu.VMEM((_CHUNK, _D), f32),
        pltpu.VMEM_SHARED((_N, _D), f32),
        pltpu.SemaphoreType.DMA,
        pltpu.SemaphoreType.DMA,
        pltpu.SemaphoreType.DMA,
        pltpu.SemaphoreType.DMA,
        pltpu.SemaphoreType.DMA,
        pltpu.SemaphoreType.DMA,
        pltpu.SemaphoreType.DMA,
        pltpu.SemaphoreType.DMA,
    ],
)
def _edge_kernel(y_hbm, src_hbm, dst_hbm, out_hbm, sidx, didx,
                 rows0, rows1, rows2, rows3, acc,
                 gs0, gs1, gs2, gs3, ss0, ss1, ss2, ss3):
    cid = lax.axis_index("c")
    sid = lax.axis_index("s")
    w = cid * _NS + sid
    base = sid * _RPS
    bufs = (rows0, rows1, rows2, rows3)
    gsems = (gs0, gs1, gs2, gs3)
    ssems = (ss0, ss1, ss2, ss3)

    @pl.loop(0, _ZCH)
    def _(i):
        @pl.loop(0, _D // 16)
        def _(c):
            rows0[i, pl.ds(c * 16, 16)] = jnp.zeros((16,), f32)

    zsrc = rows0.at[pl.ds(0, _ZCH)]

    @pl.when(sid < _NS - 1)
    def _():
        @pl.loop(0, _RPS // _ZCH)
        def _(k):
            pltpu.sync_copy(zsrc, acc.at[pl.ds(base + k * _ZCH, _ZCH)])

    @pl.when(sid == _NS - 1)
    def _():
        @pl.loop(0, _RLAST // _ZCH)
        def _(k):
            pltpu.sync_copy(zsrc, acc.at[pl.ds(base + k * _ZCH, _ZCH)])

    plsc.subcore_barrier()

    for s in range(_NSEC):
        pltpu.sync_copy(src_hbm.at[w, pl.ds(s * _SCH, _SCH)], sidx)
        pltpu.sync_copy(dst_hbm.at[w, pl.ds(s * _SCH, _SCH)], didx)

        for k in range(_NB):
            pltpu.async_copy(y_hbm.at[sidx.at[k]], bufs[k], gsems[k])

        @pl.loop(0, _SCH, step=_NB)
        def _(j):
            for k in range(_NB):
                pltpu.make_async_copy(y_hbm.at[sidx.at[j + k]], bufs[k], gsems[k]).wait()
                pltpu.async_copy(bufs[k], acc.at[didx.at[j + k]], ssems[k], add=True)

            @pl.when(j < _SCH - _NB)
            def _():
                for k in range(_NB):
                    pltpu.make_async_copy(bufs[k], acc.at[didx.at[j + k]], ssems[k]).wait()
                    pltpu.async_copy(y_hbm.at[sidx.at[j + k + _NB]], bufs[k], gsems[k])

            @pl.when(j >= _SCH - _NB)
            def _():
                for k in range(_NB):
                    pltpu.make_async_copy(bufs[k], acc.at[didx.at[j + k]], ssems[k]).wait()

    plsc.subcore_barrier()

    @pl.when(sid < _NS - 1)
    def _():
        pltpu.sync_copy(
            acc.at[pl.ds(base, _RPS)], out_hbm.at[cid, pl.ds(base, _RPS)]
        )

    @pl.when(sid == _NS - 1)
    def _():
        pltpu.sync_copy(
            acc.at[pl.ds(base, _RLAST)], out_hbm.at[cid, pl.ds(base, _RLAST)]
        )


_BLK = 1000
_GRID = _N // _BLK


def _mm(x, W, da, db):
    def body(x_ref, w_ref, da_ref, db_ref, o_ref):
        d = lax.rsqrt(da_ref[:, 0:1] + db_ref[:, 0:1] + 1.0)
        o_ref[...] = (
            jnp.dot(x_ref[...], w_ref[...],
                    preferred_element_type=f32,
                    precision=lax.Precision.HIGHEST)
            * d
        )

    return pl.pallas_call(
        body,
        grid=(_GRID,),
        in_specs=[
            pl.BlockSpec((_BLK, _D), lambda i: (i, 0)),
            pl.BlockSpec((_D, _D), lambda i: (0, 0)),
            pl.BlockSpec((_BLK, _D), lambda i: (i, 0)),
            pl.BlockSpec((_BLK, _D), lambda i: (i, 0)),
        ],
        out_specs=pl.BlockSpec((_BLK, _D), lambda i: (i, 0)),
        out_shape=jax.ShapeDtypeStruct((_N, _D), f32),
    )(x, W, da, db)


def _mid(sa, sb, y, da, db, b1, W2):
    def body(sa_ref, sb_ref, y_ref, da_ref, db_ref, b_ref, w_ref, o_ref):
        d = lax.rsqrt(da_ref[:, 0:1] + db_ref[:, 0:1] + 1.0)
        h = jnp.maximum(
            d * (sa_ref[...] + sb_ref[...] + y_ref[...]) + b_ref[...], 0.0
        )
        o_ref[...] = (
            jnp.dot(h, w_ref[...],
                    preferred_element_type=f32,
                    precision=lax.Precision.HIGHEST)
            * d
        )

    return pl.pallas_call(
        body,
        grid=(_GRID,),
        in_specs=[
            pl.BlockSpec((_BLK, _D), lambda i: (i, 0)),
            pl.BlockSpec((_BLK, _D), lambda i: (i, 0)),
            pl.BlockSpec((_BLK, _D), lambda i: (i, 0)),
            pl.BlockSpec((_BLK, _D), lambda i: (i, 0)),
            pl.BlockSpec((_BLK, _D), lambda i: (i, 0)),
            pl.BlockSpec((1, _D), lambda i: (0, 0)),
            pl.BlockSpec((_D, _D), lambda i: (0, 0)),
        ],
        out_specs=pl.BlockSpec((_BLK, _D), lambda i: (i, 0)),
        out_shape=jax.ShapeDtypeStruct((_N, _D), f32),
    )(sa, sb, y, da, db, b1, W2)


def _final(sa, sb, y, da, db, b2):
    def body(sa_ref, sb_ref, y_ref, da_ref, db_ref, b_ref, o_ref):
        d = lax.rsqrt(da_ref[:, 0:1] + db_ref[:, 0:1] + 1.0)
        o_ref[...] = d * (sa_ref[...] + sb_ref[...] + y_ref[...]) + b_ref[...]

    return pl.pallas_call(
        body,
        grid=(_GRID,),
        in_specs=[
            pl.BlockSpec((_BLK, _D), lambda i: (i, 0)),
            pl.BlockSpec((_BLK, _D), lambda i: (i, 0)),
            pl.BlockSpec((_BLK, _D), lambda i: (i, 0)),
            pl.BlockSpec((_BLK, _D), lambda i: (i, 0)),
            pl.BlockSpec((_BLK, _D), lambda i: (i, 0)),
            pl.BlockSpec((1, _D), lambda i: (0, 0)),
        ],
        out_specs=pl.BlockSpec((_BLK, _D), lambda i: (i, 0)),
        out_shape=jax.ShapeDtypeStruct((_N, _D), f32),
    )(sa, sb, y, da, db, b2)


def kernel(x, edge_index, W1, b1, W2, b2):
    src3 = edge_index[0].reshape(_NW, _NCHUNK, _CHUNK)
    dst3 = edge_index[1].reshape(_NW, _NCHUNK, _CHUNK)

    degp = _deg_kernel(dst3)
    da, db = degp[0], degp[1]

    y1 = _mm(x, W1, da, db)
    s1 = _edge_kernel(y1, src3, dst3)
    y2 = _mid(s1[0], s1[1], y1, da, db, b1.reshape(1, _D), W2)
    s2 = _edge_kernel(y2, src3, dst3)
    return _final(s2[0], s2[1], y2, da, db, b2.reshape(1, _D))


# speedup vs baseline: 26.5595x; 1.1010x over previous
import functools

import jax
import jax.numpy as jnp
from jax import lax
from jax.experimental import pallas as pl
from jax.experimental.pallas import tpu as pltpu
from jax.experimental.pallas import tpu_sc as plsc

f32 = jnp.float32

_N = 10000
_E = 320000
_D = 128

_NC = 2
_NS = 16
_NW = _NC * _NS
_EPW = _E // _NW
_CHUNK = 50
_NCHUNK = _EPW // _CHUNK
_NB = 4
_NSEC = 5
_SCH = _NCHUNK // _NSEC
_RPS = 640
_RLAST = _N - 15 * _RPS
_ZCH = 40

_mesh = plsc.VectorSubcoreMesh(core_axis_name="c", subcore_axis_name="s")


@functools.partial(
    pl.kernel,
    out_type=jax.ShapeDtypeStruct((_NC, _N, _D), f32),
    mesh=_mesh,
    scratch_types=[
        pltpu.VMEM((_NCHUNK, _CHUNK), jnp.int32),
        pltpu.VMEM((_CHUNK, _D), f32),
        pltpu.VMEM((_ZCH, _D), f32),
        pltpu.VMEM_SHARED((_N, _D), f32),
        pltpu.SemaphoreType.DMA,
    ],
)
def _deg_kernel(dst_hbm, out_hbm, didx, ones, zbuf, acc, ssem):
    cid = lax.axis_index("c")
    sid = lax.axis_index("s")
    w = cid * _NS + sid
    base = sid * _RPS

    @pl.loop(0, _CHUNK)
    def _(i):
        @pl.loop(0, _D // 16)
        def _(c):
            ones[i, pl.ds(c * 16, 16)] = jnp.ones((16,), f32)

    @pl.loop(0, _ZCH)
    def _(i):
        @pl.loop(0, _D // 16)
        def _(c):
            zbuf[i, pl.ds(c * 16, 16)] = jnp.zeros((16,), f32)

    nz = jnp.where(sid < _NS - 1, _RPS // _ZCH, _RLAST // _ZCH)

    @pl.loop(0, _RPS // _ZCH)
    def _(k):
        @pl.when(k < nz)
        def _():
            pltpu.async_copy(zbuf, acc.at[pl.ds(base + k * _ZCH, _ZCH)], ssem)

    @pl.loop(0, _RPS // _ZCH)
    def _(k):
        @pl.when(k < nz)
        def _():
            pltpu.make_async_copy(zbuf, acc.at[pl.ds(base + k * _ZCH, _ZCH)], ssem).wait()

    plsc.subcore_barrier()

    pltpu.sync_copy(dst_hbm.at[w], didx)

    @pl.loop(0, _NCHUNK)
    def _(j):
        pltpu.async_copy(ones, acc.at[didx.at[j]], ssem, add=True)

    @pl.loop(0, _NCHUNK)
    def _(j):
        pltpu.make_async_copy(ones, acc.at[didx.at[j]], ssem).wait()

    plsc.subcore_barrier()

    @pl.when(sid < _NS - 1)
    def _():
        pltpu.sync_copy(
            acc.at[pl.ds(base, _RPS)], out_hbm.at[cid, pl.ds(base, _RPS)]
        )

    @pl.when(sid == _NS - 1)
    def _():
        pltpu.sync_copy(
            acc.at[pl.ds(base, _RLAST)], out_hbm.at[cid, pl.ds(base, _RLAST)]
        )


@functools.partial(
    pl.kernel,
    out_type=jax.ShapeDtypeStruct((_NC, _N, _D), f32),
    mesh=_mesh,
    scratch_types=[
        pltpu.VMEM((_SCH, _CHUNK), jnp.int32),
        pltpu.VMEM((_SCH, _CHUNK), jnp.int32),
        pltpu.VMEM((_SCH, _CHUNK), jnp.int32),
        pltpu.VMEM((_SCH, _CHUNK), jnp.int32),
        pltpu.VMEM((_CHUNK, _D), f32),
        pltpu.VMEM((_CHUNK, _D), f32),
        pltpu.VMEM((_CHUNK, _D), f32),
        pltpu.VMEM((_CHUNK, _D), f32),
        pltpu.VMEM_SHARED((_N, _D), f32),
        pltpu.SemaphoreType.DMA,
        pltpu.SemaphoreType.DMA,
        pltpu.SemaphoreType.DMA,
        pltpu.SemaphoreType.DMA,
        pltpu.SemaphoreType.DMA,
        pltpu.SemaphoreType.DMA,
        pltpu.SemaphoreType.DMA,
        pltpu.SemaphoreType.DMA,
        pltpu.SemaphoreType.DMA,
    ],
)
def _edge_kernel(y_hbm, src_hbm, dst_hbm, out_hbm, sidxa, didxa, sidxb, didxb,
                 rows0, rows1, rows2, rows3, acc,
                 gs0, gs1, gs2, gs3, ss0, ss1, ss2, ss3, slsem):
    cid = lax.axis_index("c")
    sid = lax.axis_index("s")
    w = cid * _NS + sid
    base = sid * _RPS
    bufs = (rows0, rows1, rows2, rows3)
    gsems = (gs0, gs1, gs2, gs3)
    ssems = (ss0, ss1, ss2, ss3)
    slabs = ((sidxa, didxa), (sidxb, didxb))

    @pl.loop(0, _ZCH)
    def _(i):
        @pl.loop(0, _D // 16)
        def _(c):
            rows0[i, pl.ds(c * 16, 16)] = jnp.zeros((16,), f32)

    zsrc = rows0.at[pl.ds(0, _ZCH)]
    nz = jnp.where(sid < _NS - 1, _RPS // _ZCH, _RLAST // _ZCH)

    @pl.loop(0, _RPS // _ZCH)
    def _(k):
        @pl.when(k < nz)
        def _():
            pltpu.async_copy(zsrc, acc.at[pl.ds(base + k * _ZCH, _ZCH)], ss0)

    @pl.loop(0, _RPS // _ZCH)
    def _(k):
        @pl.when(k < nz)
        def _():
            pltpu.make_async_copy(zsrc, acc.at[pl.ds(base + k * _ZCH, _ZCH)], ss0).wait()

    plsc.subcore_barrier()

    pltpu.sync_copy(src_hbm.at[w, pl.ds(0, _SCH)], sidxa)
    pltpu.sync_copy(dst_hbm.at[w, pl.ds(0, _SCH)], didxa)

    for k in range(_NB):
        pltpu.async_copy(y_hbm.at[sidxa.at[k]], bufs[k], gsems[k])

    for s in range(_NSEC):
        sidx, didx = slabs[s % 2]
        nsidx, ndidx = slabs[(s + 1) % 2]
        if s + 1 < _NSEC:
            pltpu.async_copy(src_hbm.at[w, pl.ds((s + 1) * _SCH, _SCH)], nsidx, slsem)
            pltpu.async_copy(dst_hbm.at[w, pl.ds((s + 1) * _SCH, _SCH)], ndidx, slsem)

        @pl.loop(0, _SCH, step=_NB)
        def _(j):
            for k in range(_NB):
                pltpu.make_async_copy(y_hbm.at[sidx.at[j + k]], bufs[k], gsems[k]).wait()
                pltpu.async_copy(bufs[k], acc.at[didx.at[j + k]], ssems[k], add=True)

            @pl.when(j < _SCH - _NB)
            def _():
                for k in range(_NB):
                    pltpu.make_async_copy(bufs[k], acc.at[didx.at[j + k]], ssems[k]).wait()
                    pltpu.async_copy(y_hbm.at[sidx.at[j + k + _NB]], bufs[k], gsems[k])

            @pl.when(j >= _SCH - _NB)
            def _():
                if s + 1 < _NSEC:
                    pltpu.make_async_copy(
                        src_hbm.at[w, pl.ds((s + 1) * _SCH, _SCH)], nsidx, slsem
                    ).wait()
                    pltpu.make_async_copy(
                        dst_hbm.at[w, pl.ds((s + 1) * _SCH, _SCH)], ndidx, slsem
                    ).wait()
                    for k in range(_NB):
                        pltpu.make_async_copy(bufs[k], acc.at[didx.at[j + k]], ssems[k]).wait()
                        pltpu.async_copy(y_hbm.at[nsidx.at[k]], bufs[k], gsems[k])
                else:
                    for k in range(_NB):
                        pltpu.make_async_copy(bufs[k], acc.at[didx.at[j + k]], ssems[k]).wait()

    plsc.subcore_barrier()

    @pl.when(sid < _NS - 1)
    def _():
        pltpu.sync_copy(
            acc.at[pl.ds(base, _RPS)], out_hbm.at[cid, pl.ds(base, _RPS)]
        )

    @pl.when(sid == _NS - 1)
    def _():
        pltpu.sync_copy(
            acc.at[pl.ds(base, _RLAST)], out_hbm.at[cid, pl.ds(base, _RLAST)]
        )


_BLK = 1000
_GRID = _N // _BLK


def _mm(x, W, degp):
    def body(x_ref, w_ref, da_ref, db_ref, o_ref):
        d = lax.rsqrt(da_ref[0, :, 0:1] + db_ref[0, :, 0:1] + 1.0)
        o_ref[...] = (
            jnp.dot(x_ref[...], w_ref[...],
                    preferred_element_type=f32,
                    precision=lax.Precision.HIGHEST)
            * d
        )

    return pl.pallas_call(
        body,
        grid=(_GRID,),
        in_specs=[
            pl.BlockSpec((_BLK, _D), lambda i: (i, 0)),
            pl.BlockSpec((_D, _D), lambda i: (0, 0)),
            pl.BlockSpec((1, _BLK, _D), lambda i: (0, i, 0)),
            pl.BlockSpec((1, _BLK, _D), lambda i: (1, i, 0)),
        ],
        out_specs=pl.BlockSpec((_BLK, _D), lambda i: (i, 0)),
        out_shape=jax.ShapeDtypeStruct((_N, _D), f32),
    )(x, W, degp, degp)


def _mid(sp, y, degp, b1, W2):
    def body(sa_ref, sb_ref, y_ref, da_ref, db_ref, b_ref, w_ref, o_ref):
        d = lax.rsqrt(da_ref[0, :, 0:1] + db_ref[0, :, 0:1] + 1.0)
        h = jnp.maximum(
            d * (sa_ref[0] + sb_ref[0] + y_ref[...]) + b_ref[...], 0.0
        )
        o_ref[...] = (
            jnp.dot(h, w_ref[...],
                    preferred_element_type=f32,
                    precision=lax.Precision.HIGHEST)
            * d
        )

    return pl.pallas_call(
        body,
        grid=(_GRID,),
        in_specs=[
            pl.BlockSpec((1, _BLK, _D), lambda i: (0, i, 0)),
            pl.BlockSpec((1, _BLK, _D), lambda i: (1, i, 0)),
            pl.BlockSpec((_BLK, _D), lambda i: (i, 0)),
            pl.BlockSpec((1, _BLK, _D), lambda i: (0, i, 0)),
            pl.BlockSpec((1, _BLK, _D), lambda i: (1, i, 0)),
            pl.BlockSpec((1, _D), lambda i: (0, 0)),
            pl.BlockSpec((_D, _D), lambda i: (0, 0)),
        ],
        out_specs=pl.BlockSpec((_BLK, _D), lambda i: (i, 0)),
        out_shape=jax.ShapeDtypeStruct((_N, _D), f32),
    )(sp, sp, y, degp, degp, b1, W2)


def _final(sp, y, degp, b2):
    def body(sa_ref, sb_ref, y_ref, da_ref, db_ref, b_ref, o_ref):
        d = lax.rsqrt(da_ref[0, :, 0:1] + db_ref[0, :, 0:1] + 1.0)
        o_ref[...] = d * (sa_ref[0] + sb_ref[0] + y_ref[...]) + b_ref[...]

    return pl.pallas_call(
        body,
        grid=(_GRID,),
        in_specs=[
            pl.BlockSpec((1, _BLK, _D), lambda i: (0, i, 0)),
            pl.BlockSpec((1, _BLK, _D), lambda i: (1, i, 0)),
            pl.BlockSpec((_BLK, _D), lambda i: (i, 0)),
            pl.BlockSpec((1, _BLK, _D), lambda i: (0, i, 0)),
            pl.BlockSpec((1, _BLK, _D), lambda i: (1, i, 0)),
            pl.BlockSpec((1, _D), lambda i: (0, 0)),
        ],
        out_specs=pl.BlockSpec((_BLK, _D), lambda i: (i, 0)),
        out_shape=jax.ShapeDtypeStruct((_N, _D), f32),
    )(sp, sp, y, degp, degp, b2)


def kernel(x, edge_index, W1, b1, W2, b2):
    src3 = edge_index[0].reshape(_NW, _NCHUNK, _CHUNK)
    dst3 = edge_index[1].reshape(_NW, _NCHUNK, _CHUNK)

    degp = _deg_kernel(dst3)
    y1 = _mm(x, W1, degp)
    s1 = _edge_kernel(y1, src3, dst3)
    y2 = _mid(s1, y1, degp, b1.reshape(1, _D), W2)
    s2 = _edge_kernel(y2, src3, dst3)
    return _final(s2, y2, degp, b2.reshape(1, _D))


# speedup vs baseline: 27.0124x; 1.0171x over previous
import functools

import jax
import jax.numpy as jnp
from jax import lax
from jax.experimental import pallas as pl
from jax.experimental.pallas import tpu as pltpu
from jax.experimental.pallas import tpu_sc as plsc

f32 = jnp.float32

_N = 10000
_E = 320000
_D = 128

_NC = 2
_NS = 16
_NW = _NC * _NS
_EPW = _E // _NW
_CHUNK = 50
_NCHUNK = _EPW // _CHUNK
_NB = 4
_NSEC = 5
_SCH = _NCHUNK // _NSEC
_RPS = 640
_RLAST = _N - 15 * _RPS
_ZCH = 40

_mesh = plsc.VectorSubcoreMesh(core_axis_name="c", subcore_axis_name="s")


@functools.partial(
    pl.kernel,
    out_type=jax.ShapeDtypeStruct((_NC, _N, _D), f32),
    mesh=_mesh,
    scratch_types=[
        pltpu.VMEM((_NCHUNK, _CHUNK), jnp.int32),
        pltpu.VMEM((_CHUNK, _D), f32),
        pltpu.VMEM((_ZCH, _D), f32),
        pltpu.VMEM_SHARED((_N, _D), f32),
        pltpu.SemaphoreType.DMA,
    ],
)
def _deg_kernel(dst_hbm, out_hbm, didx, ones, zbuf, acc, ssem):
    cid = lax.axis_index("c")
    sid = lax.axis_index("s")
    w = cid * _NS + sid
    base = sid * _RPS

    @pl.loop(0, _CHUNK)
    def _(i):
        @pl.loop(0, _D // 16)
        def _(c):
            ones[i, pl.ds(c * 16, 16)] = jnp.ones((16,), f32)

    @pl.loop(0, _ZCH)
    def _(i):
        @pl.loop(0, _D // 16)
        def _(c):
            zbuf[i, pl.ds(c * 16, 16)] = jnp.zeros((16,), f32)

    nz = jnp.where(sid < _NS - 1, _RPS // _ZCH, _RLAST // _ZCH)

    @pl.loop(0, _RPS // _ZCH)
    def _(k):
        @pl.when(k < nz)
        def _():
            pltpu.async_copy(zbuf, acc.at[pl.ds(base + k * _ZCH, _ZCH)], ssem)

    @pl.loop(0, _RPS // _ZCH)
    def _(k):
        @pl.when(k < nz)
        def _():
            pltpu.make_async_copy(zbuf, acc.at[pl.ds(base + k * _ZCH, _ZCH)], ssem).wait()

    plsc.subcore_barrier()

    pltpu.sync_copy(dst_hbm.at[w], didx)

    @pl.loop(0, _NCHUNK)
    def _(j):
        pltpu.async_copy(ones, acc.at[didx.at[j]], ssem, add=True)

    @pl.loop(0, _NCHUNK)
    def _(j):
        pltpu.make_async_copy(ones, acc.at[didx.at[j]], ssem).wait()

    plsc.subcore_barrier()

    @pl.when(sid < _NS - 1)
    def _():
        pltpu.sync_copy(
            acc.at[pl.ds(base, _RPS)], out_hbm.at[cid, pl.ds(base, _RPS)]
        )

    @pl.when(sid == _NS - 1)
    def _():
        pltpu.sync_copy(
            acc.at[pl.ds(base, _RLAST)], out_hbm.at[cid, pl.ds(base, _RLAST)]
        )


@functools.partial(
    pl.kernel,
    out_type=jax.ShapeDtypeStruct((_NC, _N, _D), f32),
    mesh=_mesh,
    scratch_types=[
        pltpu.VMEM((_SCH, _CHUNK), jnp.int32),
        pltpu.VMEM((_SCH, _CHUNK), jnp.int32),
        pltpu.VMEM((_SCH, _CHUNK), jnp.int32),
        pltpu.VMEM((_SCH, _CHUNK), jnp.int32),
        pltpu.VMEM((_CHUNK, _D), f32),
        pltpu.VMEM((_CHUNK, _D), f32),
        pltpu.VMEM((_CHUNK, _D), f32),
        pltpu.VMEM((_CHUNK, _D), f32),
        pltpu.VMEM_SHARED((_N, _D), f32),
        pltpu.SemaphoreType.DMA,
        pltpu.SemaphoreType.DMA,
        pltpu.SemaphoreType.DMA,
        pltpu.SemaphoreType.DMA,
        pltpu.SemaphoreType.DMA,
        pltpu.SemaphoreType.DMA,
        pltpu.SemaphoreType.DMA,
        pltpu.SemaphoreType.DMA,
        pltpu.SemaphoreType.DMA,
    ],
)
def _edge_kernel(y_hbm, src_hbm, dst_hbm, out_hbm, sidxa, didxa, sidxb, didxb,
                 rows0, rows1, rows2, rows3, acc,
                 gs0, gs1, gs2, gs3, ss0, ss1, ss2, ss3, slsem):
    cid = lax.axis_index("c")
    sid = lax.axis_index("s")
    w = cid * _NS + sid
    base = sid * _RPS
    bufs = (rows0, rows1, rows2, rows3)
    gsems = (gs0, gs1, gs2, gs3)
    ssems = (ss0, ss1, ss2, ss3)
    slabs = ((sidxa, didxa), (sidxb, didxb))

    @pl.loop(0, _ZCH)
    def _(i):
        @pl.loop(0, _D // 16)
        def _(c):
            rows0[i, pl.ds(c * 16, 16)] = jnp.zeros((16,), f32)

    zsrc = rows0.at[pl.ds(0, _ZCH)]
    nz = jnp.where(sid < _NS - 1, _RPS // _ZCH, _RLAST // _ZCH)

    @pl.loop(0, _RPS // _ZCH)
    def _(k):
        @pl.when(k < nz)
        def _():
            pltpu.async_copy(zsrc, acc.at[pl.ds(base + k * _ZCH, _ZCH)], ss0)

    @pl.loop(0, _RPS // _ZCH)
    def _(k):
        @pl.when(k < nz)
        def _():
            pltpu.make_async_copy(zsrc, acc.at[pl.ds(base + k * _ZCH, _ZCH)], ss0).wait()

    plsc.subcore_barrier()

    pltpu.sync_copy(src_hbm.at[w, pl.ds(0, _SCH)], sidxa)
    pltpu.sync_copy(dst_hbm.at[w, pl.ds(0, _SCH)], didxa)

    for k in range(_NB):
        pltpu.async_copy(y_hbm.at[sidxa.at[k]], bufs[k], gsems[k])

    for s in range(_NSEC):
        sidx, didx = slabs[s % 2]
        nsidx, ndidx = slabs[(s + 1) % 2]
        if s + 1 < _NSEC:
            pltpu.async_copy(src_hbm.at[w, pl.ds((s + 1) * _SCH, _SCH)], nsidx, slsem)
            pltpu.async_copy(dst_hbm.at[w, pl.ds((s + 1) * _SCH, _SCH)], ndidx, slsem)

        @pl.loop(0, _SCH, step=_NB)
        def _(j):
            for k in range(_NB):
                pltpu.make_async_copy(y_hbm.at[sidx.at[j + k]], bufs[k], gsems[k]).wait()
                pltpu.async_copy(bufs[k], acc.at[didx.at[j + k]], ssems[k], add=True)

            @pl.when(j < _SCH - _NB)
            def _():
                for k in range(_NB):
                    pltpu.make_async_copy(bufs[k], acc.at[didx.at[j + k]], ssems[k]).wait()
                    pltpu.async_copy(y_hbm.at[sidx.at[j + k + _NB]], bufs[k], gsems[k])

            @pl.when(j >= _SCH - _NB)
            def _():
                if s + 1 < _NSEC:
                    pltpu.make_async_copy(
                        src_hbm.at[w, pl.ds((s + 1) * _SCH, _SCH)], nsidx, slsem
                    ).wait()
                    pltpu.make_async_copy(
                        dst_hbm.at[w, pl.ds((s + 1) * _SCH, _SCH)], ndidx, slsem
                    ).wait()
                    for k in range(_NB):
                        pltpu.make_async_copy(bufs[k], acc.at[didx.at[j + k]], ssems[k]).wait()
                        pltpu.async_copy(y_hbm.at[nsidx.at[k]], bufs[k], gsems[k])
                else:
                    for k in range(_NB):
                        pltpu.make_async_copy(bufs[k], acc.at[didx.at[j + k]], ssems[k]).wait()

    plsc.subcore_barrier()

    @pl.when(sid < _NS - 1)
    def _():
        pltpu.sync_copy(
            acc.at[pl.ds(base, _RPS)], out_hbm.at[cid, pl.ds(base, _RPS)]
        )

    @pl.when(sid == _NS - 1)
    def _():
        pltpu.sync_copy(
            acc.at[pl.ds(base, _RLAST)], out_hbm.at[cid, pl.ds(base, _RLAST)]
        )


_BLK = 1000
_GRID = _N // _BLK


def _mm(x, W):
    def body(x_ref, w_ref, o_ref):
        o_ref[...] = jnp.dot(
            x_ref[...], w_ref[...],
            preferred_element_type=f32, precision=lax.Precision.HIGHEST,
        )

    return pl.pallas_call(
        body,
        grid=(_GRID,),
        in_specs=[
            pl.BlockSpec((_BLK, _D), lambda i: (i, 0)),
            pl.BlockSpec((_D, _D), lambda i: (0, 0)),
        ],
        out_specs=pl.BlockSpec((_BLK, _D), lambda i: (i, 0)),
        out_shape=jax.ShapeDtypeStruct((_N, _D), f32),
    )(x, W)


def _dscale(xw, degp):
    def body(xw_ref, da_ref, db_ref, y_ref, dn_ref):
        d = lax.rsqrt(da_ref[0, :, 0:1] + db_ref[0, :, 0:1] + 1.0)
        y_ref[...] = xw_ref[...] * d
        dn_ref[...] = jnp.broadcast_to(d, (_BLK, 8))

    return pl.pallas_call(
        body,
        grid=(_GRID,),
        in_specs=[
            pl.BlockSpec((_BLK, _D), lambda i: (i, 0)),
            pl.BlockSpec((1, _BLK, _D), lambda i: (0, i, 0)),
            pl.BlockSpec((1, _BLK, _D), lambda i: (1, i, 0)),
        ],
        out_specs=[
            pl.BlockSpec((_BLK, _D), lambda i: (i, 0)),
            pl.BlockSpec((_BLK, 8), lambda i: (i, 0)),
        ],
        out_shape=[
            jax.ShapeDtypeStruct((_N, _D), f32),
            jax.ShapeDtypeStruct((_N, 8), f32),
        ],
    )(xw, degp, degp)


def _mid(sp, y, dn, b1, W2):
    def body(sa_ref, sb_ref, y_ref, dn_ref, b_ref, w_ref, o_ref):
        d = dn_ref[:, 0:1]
        h = jnp.maximum(
            d * (sa_ref[0] + sb_ref[0] + y_ref[...]) + b_ref[...], 0.0
        )
        o_ref[...] = (
            jnp.dot(h, w_ref[...],
                    preferred_element_type=f32,
                    precision=lax.Precision.HIGHEST)
            * d
        )

    return pl.pallas_call(
        body,
        grid=(_GRID,),
        in_specs=[
            pl.BlockSpec((1, _BLK, _D), lambda i: (0, i, 0)),
            pl.BlockSpec((1, _BLK, _D), lambda i: (1, i, 0)),
            pl.BlockSpec((_BLK, _D), lambda i: (i, 0)),
            pl.BlockSpec((_BLK, 8), lambda i: (i, 0)),
            pl.BlockSpec((1, _D), lambda i: (0, 0)),
            pl.BlockSpec((_D, _D), lambda i: (0, 0)),
        ],
        out_specs=pl.BlockSpec((_BLK, _D), lambda i: (i, 0)),
        out_shape=jax.ShapeDtypeStruct((_N, _D), f32),
    )(sp, sp, y, dn, b1, W2)


def _final(sp, y, dn, b2):
    def body(sa_ref, sb_ref, y_ref, dn_ref, b_ref, o_ref):
        d = dn_ref[:, 0:1]
        o_ref[...] = d * (sa_ref[0] + sb_ref[0] + y_ref[...]) + b_ref[...]

    return pl.pallas_call(
        body,
        grid=(_GRID,),
        in_specs=[
            pl.BlockSpec((1, _BLK, _D), lambda i: (0, i, 0)),
            pl.BlockSpec((1, _BLK, _D), lambda i: (1, i, 0)),
            pl.BlockSpec((_BLK, _D), lambda i: (i, 0)),
            pl.BlockSpec((_BLK, 8), lambda i: (i, 0)),
            pl.BlockSpec((1, _D), lambda i: (0, 0)),
        ],
        out_specs=pl.BlockSpec((_BLK, _D), lambda i: (i, 0)),
        out_shape=jax.ShapeDtypeStruct((_N, _D), f32),
    )(sp, sp, y, dn, b2)


def kernel(x, edge_index, W1, b1, W2, b2):
    src3 = edge_index[0].reshape(_NW, _NCHUNK, _CHUNK)
    dst3 = edge_index[1].reshape(_NW, _NCHUNK, _CHUNK)

    degp = _deg_kernel(dst3)
    xw1 = _mm(x, W1)
    y1, dn = _dscale(xw1, degp)
    s1 = _edge_kernel(y1, src3, dst3)
    y2 = _mid(s1, y1, dn, b1.reshape(1, _D), W2)
    s2 = _edge_kernel(y2, src3, dst3)
    return _final(s2, y2, dn, b2.reshape(1, _D))
